# baseline (speedup 1.0000x reference)
import functools

import jax
import jax.numpy as jnp
from jax import lax
from jax.experimental import pallas as pl
from jax.experimental.pallas import tpu as pltpu
from jax.experimental.pallas import tpu_sc as plsc

N = 10000
E = 160000
C = 128
K5 = 125

NC = 2
NS = 16
NW = NC * NS
NP = 10240
NPT = NP // NS
ZB = 64

CBL = 16
NCH_L = E // CBL
NIT_L = (NCH_L + NW - 1) // NW

CBS = 64
NCH_S = E // CBS
NIT_S = (NCH_S + NW - 1) // NW


def _basis_body(ei_ref, at_ref, gidx_ref, bw_ref):
    src = ei_ref[0, :]
    p0 = at_ref[0, :] * 4.0
    p1 = at_ref[1, :] * 4.0
    p2 = at_ref[2, :] * 4.0
    fl0 = jnp.floor(p0)
    fl1 = jnp.floor(p1)
    fl2 = jnp.floor(p2)
    fr = (p0 - fl0, p1 - fl1, p2 - fl2)
    fli = (fl0.astype(jnp.int32), fl1.astype(jnp.int32), fl2.astype(jnp.int32))
    off = (1, 5, 25)
    for s in range(8):
        b = None
        wi = None
        for d in range(3):
            bit = (s >> d) & 1
            v = fr[d] if bit == 1 else (1.0 - fr[d])
            b = v if b is None else b * v
            idx = (fli[d] + bit) % 5
            t = idx * off[d]
            wi = t if wi is None else wi + t
        gidx_ref[s, :] = wi * N + src
        bw_ref[s, :] = b


def _basis(edge_index, attr_t):
    return pl.pallas_call(
        _basis_body,
        out_shape=(
            jax.ShapeDtypeStruct((8, E), jnp.int32),
            jax.ShapeDtypeStruct((8, E), jnp.float32),
        ),
    )(edge_index, attr_t)


BN_N = 1000


def _ymm_body(x_ref, w_ref, y_ref):
    y_ref[0] = jnp.dot(x_ref[...], w_ref[0],
                       preferred_element_type=jnp.float32)


def _ymm(x, w):
    nt = N // BN_N
    y = pl.pallas_call(
        _ymm_body,
        grid=(nt, K5),
        in_specs=[
            pl.BlockSpec((BN_N, C), lambda n, k: (n, 0)),
            pl.BlockSpec((1, C, C), lambda n, k: (k, 0, 0)),
        ],
        out_specs=pl.BlockSpec((1, BN_N, C), lambda n, k: (k, n, 0)),
        out_shape=jax.ShapeDtypeStruct((K5, N, C), jnp.float32),
    )(x, w)
    return y.reshape(K5 * N, C)


def _zero_fill(zb_v, width):
    def _zrow(r, _):
        for cc in range(width // 16):
            zb_v[r, pl.ds(cc * 16, 16)] = jnp.zeros((16,), jnp.float32)
        return 0

    lax.fori_loop(0, ZB, _zrow, 0)


def _zero_spmem(zb_v, sh, s, width):
    for r in range(NPT // ZB):
        pltpu.sync_copy(zb_v.at[0:ZB, 0:width],
                        sh.at[pl.ds(s * NPT + r * ZB, ZB)])


HP = NP // NC
HPT = HP // NS


def _sc_short_body(x_hbm, src_hbm, dst_hbm, aggx_hbm, cnt_hbm,
                   src_v, dst_v, ldst_v, rows_v, ones_v, zb_v,
                   aggx_sh, cnt_sh, sem):
    c = lax.axis_index("c")
    s = lax.axis_index("s")

    _zero_fill(zb_v, C)

    def _orow(r, _):
        for cc in range(8):
            ones_v[r, pl.ds(cc * 16, 16)] = jnp.ones((16,), jnp.float32)
        return 0

    lax.fori_loop(0, CBS, _orow, 0)

    for r in range(HPT // ZB):
        pltpu.sync_copy(zb_v.at[0:ZB, :],
                        aggx_sh.at[pl.ds(s * HPT + r * ZB, ZB)])
        pltpu.sync_copy(zb_v.at[0:ZB, :],
                        cnt_sh.at[pl.ds(s * HPT + r * ZB, ZB)])

    @pl.when(s == 0)
    def _():
        pltpu.sync_copy(zb_v.at[0:16, :], aggx_sh.at[pl.ds(HP, 16)])
        pltpu.sync_copy(zb_v.at[0:16, :], cnt_sh.at[pl.ds(HP, 16)])

    plsc.subcore_barrier()
    lo = c * HP

    def _chunk(j, _):
        cid = s + j * NS

        @pl.when(cid < NCH_S)
        def _():
            base = cid * CBS
            pltpu.sync_copy(src_hbm.at[pl.ds(base, CBS)], src_v)
            pltpu.sync_copy(dst_hbm.at[pl.ds(base, CBS)], dst_v)
            for r in range(CBS // 16):
                sl = pl.ds(r * 16, 16)
                d = dst_v[sl] - lo
                ok = (d >= 0) & (d < HP)
                ldst_v[sl] = jnp.where(ok, d, HP)
            pltpu.async_copy(x_hbm.at[src_v], rows_v, sem).wait()
            pltpu.sync_copy(rows_v, aggx_sh.at[ldst_v], add=True)
            pltpu.sync_copy(ones_v, cnt_sh.at[ldst_v], add=True)

        return 0

    lax.fori_loop(0, NIT_S2, _chunk, 0)
    plsc.subcore_barrier()

    pltpu.sync_copy(aggx_sh.at[pl.ds(s * HPT, HPT)],
                    aggx_hbm.at[c, pl.ds(s * HPT, HPT)])
    pltpu.sync_copy(cnt_sh.at[pl.ds(s * HPT, HPT)],
                    cnt_hbm.at[c, pl.ds(s * HPT, HPT)])


NIT_S2 = (NCH_S + NS - 1) // NS


def _sc_short(x, src, dst):
    mesh = plsc.VectorSubcoreMesh(core_axis_name="c", subcore_axis_name="s")
    f = functools.partial(
        pl.kernel,
        mesh=mesh,
        out_type=(
            jax.ShapeDtypeStruct((NC, HP, C), jnp.float32),
            jax.ShapeDtypeStruct((NC, HP, C), jnp.float32),
        ),
        scratch_types=[
            pltpu.VMEM((CBS,), jnp.int32),
            pltpu.VMEM((CBS,), jnp.int32),
            pltpu.VMEM((CBS,), jnp.int32),
            pltpu.VMEM((CBS, C), jnp.float32),
            pltpu.VMEM((CBS, C), jnp.float32),
            pltpu.VMEM((ZB, C), jnp.float32),
            pltpu.VMEM_SHARED((HP + 16, C), jnp.float32),
            pltpu.VMEM_SHARED((HP + 16, C), jnp.float32),
            pltpu.SemaphoreType.DMA,
        ],
    )(_sc_short_body)
    return f(x, src, dst)


def _sc_layer_body(y_hbm, gidx_hbm, bw_hbm, dst_hbm, agg_hbm,
                   gidx_v, bw_v, dst_v, rows_v, msg_v, zb_v, agg_sh, sem):
    c = lax.axis_index("c")
    s = lax.axis_index("s")
    wid = s * NC + c

    _zero_fill(zb_v, C)
    _zero_spmem(zb_v, agg_sh, s, C)
    plsc.subcore_barrier()

    def _chunk(j, _):
        cid = wid + j * NW

        @pl.when(cid < NCH_L)
        def _():
            base = cid * CBL
            pltpu.sync_copy(gidx_hbm.at[pl.ds(base * 8, CBL * 8)], gidx_v)
            pltpu.sync_copy(bw_hbm.at[pl.ds(base * 8, CBL * 8)],
                            bw_v.at[pl.ds(0, CBL * 8)])
            pltpu.sync_copy(dst_hbm.at[pl.ds(base, CBL)], dst_v)
            pltpu.async_copy(y_hbm.at[gidx_v], rows_v, sem).wait()

            def _edge(e, _):
                eb = e * 8
                b = [bw_v[pl.ds(eb + t, 16)][0] for t in range(8)]
                for cc in range(8):
                    sl = pl.ds(cc * 16, 16)
                    acc = b[0] * rows_v[eb, sl]
                    for t in range(1, 8):
                        acc = acc + b[t] * rows_v[eb + t, sl]
                    msg_v[e, sl] = acc
                return 0

            lax.fori_loop(0, CBL, _edge, 0)
            pltpu.sync_copy(msg_v, agg_sh.at[dst_v], add=True)

        return 0

    lax.fori_loop(0, NIT_L, _chunk, 0)
    plsc.subcore_barrier()
    pltpu.sync_copy(agg_sh.at[pl.ds(s * NPT, NPT)],
                    agg_hbm.at[c, pl.ds(s * NPT, NPT)])


def _sc_layer(y, gidx_ef, bw_ef, dst):
    mesh = plsc.VectorSubcoreMesh(core_axis_name="c", subcore_axis_name="s")
    f = functools.partial(
        pl.kernel,
        mesh=mesh,
        out_type=jax.ShapeDtypeStruct((NC, NP, C), jnp.float32),
        scratch_types=[
            pltpu.VMEM((CBL * 8,), jnp.int32),
            pltpu.VMEM((CBL * 8 + 16,), jnp.float32),
            pltpu.VMEM((CBL,), jnp.int32),
            pltpu.VMEM((CBL * 8, C), jnp.float32),
            pltpu.VMEM((CBL, C), jnp.float32),
            pltpu.VMEM((ZB, C), jnp.float32),
            pltpu.VMEM_SHARED((NP, C), jnp.float32),
            pltpu.SemaphoreType.DMA,
        ],
    )(_sc_layer_body)
    return f(y, gidx_ef, bw_ef, dst)


def _bn(pre, g, be):
    mu = jnp.mean(pre, axis=0)
    d = pre - mu
    var = jnp.mean(d * d, axis=0)
    return g * d / jnp.sqrt(var + 1e-5) + be


def _elu(v):
    return jnp.where(v > 0.0, v, jnp.exp(jnp.minimum(v, 0.0)) - 1.0)


def _cnt_full(cnt_ref):
    return jnp.concatenate([cnt_ref[0], cnt_ref[1]], axis=0)[0:N, :]


def _post1_body(agg_ref, cnt_ref, x_ref, root_ref, b_ref, g_ref, be_ref,
                h_ref):
    cnt = _cnt_full(cnt_ref)
    agg = (agg_ref[0][0:N, :] + agg_ref[1][0:N, :]) / jnp.maximum(cnt, 1.0)
    pre = agg + jnp.dot(x_ref[...], root_ref[...],
                        preferred_element_type=jnp.float32) + b_ref[...]
    h_ref[...] = _elu(_bn(pre, g_ref[...], be_ref[...]))


def _post1(agg, cnt, x, root, b, g, be):
    return pl.pallas_call(
        _post1_body,
        out_shape=jax.ShapeDtypeStruct((N, C), jnp.float32),
    )(agg, cnt, x, root, b, g, be)


def _post2_body(agg_ref, aggx_ref, cnt_ref, h_ref, x_ref,
                root2_ref, b2_ref, g2_ref, be2_ref,
                ws_ref, roots_ref, bs_ref, gs_ref, bes_ref, out_ref):
    cnt = jnp.maximum(_cnt_full(cnt_ref), 1.0)
    agg = (agg_ref[0][0:N, :] + agg_ref[1][0:N, :]) / cnt
    left_pre = agg + jnp.dot(h_ref[...], root2_ref[...],
                             preferred_element_type=jnp.float32) + b2_ref[...]
    left = _bn(left_pre, g2_ref[...], be2_ref[...])
    aggx = jnp.concatenate([aggx_ref[0], aggx_ref[1]], axis=0)[0:N, :] / cnt
    short_pre = (jnp.dot(aggx, ws_ref[...], preferred_element_type=jnp.float32)
                 + jnp.dot(x_ref[...], roots_ref[...],
                           preferred_element_type=jnp.float32) + bs_ref[...])
    short = _bn(short_pre, gs_ref[...], bes_ref[...])
    out_ref[...] = _elu(left + short)


def _post2(agg, aggx, cnt, h, x, root2, b2, g2, be2, ws, roots, bs, gs, bes):
    return pl.pallas_call(
        _post2_body,
        out_shape=jax.ShapeDtypeStruct((N, C), jnp.float32),
    )(agg, aggx, cnt, h, x, root2, b2, g2, be2, ws, roots, bs, gs, bes)


def kernel(x, edge_index, edge_attr, W1, root1, b1, g1, be1,
           W2, root2, b2, g2, be2, Ws, roots, bs, gs, bes):
    edge_index = edge_index.astype(jnp.int32)
    src = edge_index[0]
    dst = edge_index[1]
    attr_t = edge_attr.T

    gidx, bw = _basis(edge_index, attr_t)
    gidx_ef = gidx.T.reshape(E * 8)
    bw_ef = bw.T.reshape(E * 8)

    y1 = _ymm(x, W1)
    aggx, cnt = _sc_short(x, src, dst)
    agg1 = _sc_layer(y1, gidx_ef, bw_ef, dst)
    h = _post1(agg1, cnt, x, root1, b1, g1, be1)
    y2 = _ymm(h, W2)
    agg2 = _sc_layer(y2, gidx_ef, bw_ef, dst)
    return _post2(agg2, aggx, cnt, h, x, root2, b2, g2, be2,
                  Ws[0], roots, bs, gs, bes)


# speedup vs baseline: 1.0642x; 1.0642x over previous
import functools

import jax
import jax.numpy as jnp
from jax import lax
from jax.experimental import pallas as pl
from jax.experimental.pallas import tpu as pltpu
from jax.experimental.pallas import tpu_sc as plsc

N = 10000
E = 160000
C = 128
K5 = 125

NC = 2
NS = 16
NW = NC * NS
NP = 10240
NPT = NP // NS
ZB = 16

CBL = 16
EPT = E // NW
NCHT = 316
EPTP = NCHT * CBL
NQUAD = NCHT // 4

PIPELINED = True

CBS = 64
NCH_S = E // CBS


def _basis_body(ei_ref, at_ref, gidx_ref, bw_ref):
    src = ei_ref[0, :]
    p0 = at_ref[0, :] * 4.0
    p1 = at_ref[1, :] * 4.0
    p2 = at_ref[2, :] * 4.0
    fl0 = jnp.floor(p0)
    fl1 = jnp.floor(p1)
    fl2 = jnp.floor(p2)
    fr = (p0 - fl0, p1 - fl1, p2 - fl2)
    fli = (fl0.astype(jnp.int32), fl1.astype(jnp.int32), fl2.astype(jnp.int32))
    off = (1, 5, 25)
    for s in range(8):
        b = None
        wi = None
        for d in range(3):
            bit = (s >> d) & 1
            v = fr[d] if bit == 1 else (1.0 - fr[d])
            b = v if b is None else b * v
            idx = (fli[d] + bit) % 5
            t = idx * off[d]
            wi = t if wi is None else wi + t
        gidx_ref[s, :] = wi * N + src
        bw_ref[s, :] = b


def _basis(edge_index, attr_t):
    return pl.pallas_call(
        _basis_body,
        out_shape=(
            jax.ShapeDtypeStruct((8, E), jnp.int32),
            jax.ShapeDtypeStruct((8, E), jnp.float32),
        ),
    )(edge_index, attr_t)


BN_N = 1000


def _ymm_body(x_ref, w_ref, y_ref):
    y_ref[0] = jnp.dot(x_ref[...], w_ref[0],
                       preferred_element_type=jnp.float32)


def _ymm(x, w):
    nt = N // BN_N
    y = pl.pallas_call(
        _ymm_body,
        grid=(nt, K5),
        in_specs=[
            pl.BlockSpec((BN_N, C), lambda n, k: (n, 0)),
            pl.BlockSpec((1, C, C), lambda n, k: (k, 0, 0)),
        ],
        out_specs=pl.BlockSpec((1, BN_N, C), lambda n, k: (k, n, 0)),
        out_shape=jax.ShapeDtypeStruct((K5, N, C), jnp.float32),
    )(x.astype(jnp.bfloat16), w.astype(jnp.bfloat16))
    return y.reshape(K5 * N, C)


def _zero_fill(zb_v, width):
    def _zrow(r, _):
        for cc in range(width // 16):
            zb_v[r, pl.ds(cc * 16, 16)] = jnp.zeros((16,), jnp.float32)
        return 0

    lax.fori_loop(0, ZB, _zrow, 0)


def _zero_spmem(zb_v, sh, s, width):
    for r in range(NPT // ZB):
        pltpu.sync_copy(zb_v.at[0:ZB, 0:width],
                        sh.at[pl.ds(s * NPT + r * ZB, ZB)])


HP = NP // NC
HPT = HP // NS


def _sc_short_body(x_hbm, src_hbm, dst_hbm, aggx_hbm, cnt_hbm,
                   src_v, dst_v, ldst_v, rows_v, ones_v, zb_v,
                   aggx_sh, cnt_sh, sem):
    c = lax.axis_index("c")
    s = lax.axis_index("s")

    _zero_fill(zb_v, C)

    def _orow(r, _):
        for cc in range(8):
            ones_v[r, pl.ds(cc * 16, 16)] = jnp.ones((16,), jnp.float32)
        return 0

    lax.fori_loop(0, CBS, _orow, 0)

    for r in range(HPT // ZB):
        pltpu.sync_copy(zb_v.at[0:ZB, :],
                        aggx_sh.at[pl.ds(s * HPT + r * ZB, ZB)])
        pltpu.sync_copy(zb_v.at[0:ZB, :],
                        cnt_sh.at[pl.ds(s * HPT + r * ZB, ZB)])

    @pl.when(s == 0)
    def _():
        pltpu.sync_copy(zb_v.at[0:16, :], aggx_sh.at[pl.ds(HP, 16)])
        pltpu.sync_copy(zb_v.at[0:16, :], cnt_sh.at[pl.ds(HP, 16)])

    plsc.subcore_barrier()
    lo = c * HP

    def _chunk(j, _):
        cid = s + j * NS

        @pl.when(cid < NCH_S)
        def _():
            base = cid * CBS
            pltpu.sync_copy(src_hbm.at[pl.ds(base, CBS)], src_v)
            pltpu.sync_copy(dst_hbm.at[pl.ds(base, CBS)], dst_v)
            for r in range(CBS // 16):
                sl = pl.ds(r * 16, 16)
                d = dst_v[sl] - lo
                ok = (d >= 0) & (d < HP)
                ldst_v[sl] = jnp.where(ok, d, HP)
            pltpu.async_copy(x_hbm.at[src_v], rows_v, sem).wait()
            pltpu.sync_copy(rows_v, aggx_sh.at[ldst_v], add=True)
            pltpu.sync_copy(ones_v, cnt_sh.at[ldst_v], add=True)

        return 0

    lax.fori_loop(0, NIT_S2, _chunk, 0)
    plsc.subcore_barrier()

    pltpu.sync_copy(aggx_sh.at[pl.ds(s * HPT, HPT)],
                    aggx_hbm.at[c, pl.ds(s * HPT, HPT)])
    pltpu.sync_copy(cnt_sh.at[pl.ds(s * HPT, HPT)],
                    cnt_hbm.at[c, pl.ds(s * HPT, HPT)])


NIT_S2 = (NCH_S + NS - 1) // NS


def _sc_short(x, src, dst):
    mesh = plsc.VectorSubcoreMesh(core_axis_name="c", subcore_axis_name="s")
    f = functools.partial(
        pl.kernel,
        mesh=mesh,
        out_type=(
            jax.ShapeDtypeStruct((NC, HP, C), jnp.float32),
            jax.ShapeDtypeStruct((NC, HP, C), jnp.float32),
        ),
        scratch_types=[
            pltpu.VMEM((CBS,), jnp.int32),
            pltpu.VMEM((CBS,), jnp.int32),
            pltpu.VMEM((CBS,), jnp.int32),
            pltpu.VMEM((CBS, C), jnp.float32),
            pltpu.VMEM((CBS, C), jnp.float32),
            pltpu.VMEM((ZB, C), jnp.float32),
            pltpu.VMEM_SHARED((HP + 16, C), jnp.float32),
            pltpu.VMEM_SHARED((HP + 16, C), jnp.float32),
            pltpu.SemaphoreType.DMA,
        ],
    )(_sc_short_body)
    return f(x, src, dst)


def _sc_layer_body(y_hbm, gidx_hbm, bw_hbm, dst_hbm, agg_hbm,
                   gidx_v, bw_v, dstm_v, rows_v, msg_v, zb_v, agg_sh,
                   semg0, semg1, semm0, semm1):
    c = lax.axis_index("c")
    s = lax.axis_index("s")
    wid = s * NC + c

    _zero_fill(zb_v, C)
    _zero_spmem(zb_v, agg_sh, s, C)
    plsc.subcore_barrier()

    gb = wid * (NCHT * 128)
    db = wid * (NCHT * 16)
    semg = (semg0, semg1)
    semm = (semm0, semm1)

    def _meta_copies(m, j0):
        return [
            pltpu.make_async_copy(gidx_hbm.at[pl.ds(gb + j0 * 128, 256)],
                                  gidx_v.at[m], semm[m]),
            pltpu.make_async_copy(bw_hbm.at[pl.ds(gb + j0 * 128, 128)],
                                  bw_v.at[m, 0], semm[m]),
            pltpu.make_async_copy(bw_hbm.at[pl.ds(gb + j0 * 128 + 128, 128)],
                                  bw_v.at[m, 1], semm[m]),
            pltpu.make_async_copy(dst_hbm.at[pl.ds(db + j0 * 16, 16)],
                                  dstm_v.at[m, 0], semm[m]),
            pltpu.make_async_copy(dst_hbm.at[pl.ds(db + j0 * 16 + 16, 16)],
                                  dstm_v.at[m, 1], semm[m]),
        ]

    def _meta_start(m, j0):
        for cp in _meta_copies(m, j0):
            cp.start()

    def _meta_wait(m):
        for cp in _meta_copies(m, 0):
            cp.wait()

    def _gather_start(m, jj, p):
        pltpu.async_copy(y_hbm.at[gidx_v.at[m, pl.ds(jj * 128, 128)]],
                         rows_v.at[p], semg[p])

    def _gather_wait(m, jj, p):
        pltpu.make_async_copy(y_hbm.at[gidx_v.at[m, pl.ds(jj * 128, 128)]],
                              rows_v.at[p], semg[p]).wait()

    def _combine(m, jj, p):
        def _epair(e2, _):
            v = bw_v[m, jj, pl.ds(e2 * 16, 16)]
            for half in range(2):
                eb = e2 * 16 + half * 8
                for cc in range(8):
                    sl = pl.ds(cc * 16, 16)
                    acc = v[half * 8] * rows_v[p, eb, sl]
                    for t in range(1, 8):
                        acc = acc + v[half * 8 + t] * rows_v[p, eb + t, sl]
                    msg_v[e2 * 2 + half, sl] = acc
            return 0

        lax.fori_loop(0, CBL // 2, _epair, 0)
        pltpu.sync_copy(msg_v, agg_sh.at[dstm_v.at[m, jj]], add=True)

    def _quad(q, _):
        nx0 = lax.rem(4 * q + 4, NCHT)
        nx1 = lax.rem(4 * q + 6, NCHT)
        _gather_wait(0, 0, 0)
        _combine(0, 0, 0)
        _gather_start(1, 0, 0)
        _gather_wait(0, 1, 1)
        _combine(0, 1, 1)
        _meta_start(0, nx0)
        _gather_start(1, 1, 1)
        _gather_wait(1, 0, 0)
        _combine(1, 0, 0)
        _meta_wait(0)
        _gather_start(0, 0, 0)
        _gather_wait(1, 1, 1)
        _combine(1, 1, 1)
        _meta_start(1, nx1)
        _meta_wait(1)
        _gather_start(0, 1, 1)
        return 0

    def _quad_serial(q, _):
        nx0 = lax.rem(4 * q + 4, NCHT)
        nx1 = lax.rem(4 * q + 6, NCHT)
        for m, jj in ((0, 0), (0, 1), (1, 0), (1, 1)):
            _gather_start(m, jj, 0)
            _gather_wait(m, jj, 0)
            _combine(m, jj, 0)
        _meta_start(0, nx0)
        _meta_wait(0)
        _meta_start(1, nx1)
        _meta_wait(1)
        return 0

    _meta_start(0, 0)
    _meta_start(1, 2)
    _meta_wait(0)
    _meta_wait(1)
    if PIPELINED:
        _gather_start(0, 0, 0)
        _gather_start(0, 1, 1)
        lax.fori_loop(0, NQUAD, _quad, 0)
        _gather_wait(0, 0, 0)
        _gather_wait(0, 1, 1)
    else:
        lax.fori_loop(0, NQUAD, _quad_serial, 0)

    plsc.subcore_barrier()
    pltpu.sync_copy(agg_sh.at[pl.ds(s * NPT, NPT)],
                    agg_hbm.at[c, pl.ds(s * NPT, NPT)])


def _sc_layer(y, gidx_p, bw_p, dst_p):
    mesh = plsc.VectorSubcoreMesh(core_axis_name="c", subcore_axis_name="s")
    f = functools.partial(
        pl.kernel,
        mesh=mesh,
        out_type=jax.ShapeDtypeStruct((NC, NP, C), jnp.float32),
        scratch_types=[
            pltpu.VMEM((2, 256), jnp.int32),
            pltpu.VMEM((2, 2, 128), jnp.float32),
            pltpu.VMEM((2, 2, 16), jnp.int32),
            pltpu.VMEM((2, 128, C), jnp.float32),
            pltpu.VMEM((CBL, C), jnp.float32),
            pltpu.VMEM((ZB, C), jnp.float32),
            pltpu.VMEM_SHARED((NP, C), jnp.float32),
            pltpu.SemaphoreType.DMA,
            pltpu.SemaphoreType.DMA,
            pltpu.SemaphoreType.DMA,
            pltpu.SemaphoreType.DMA,
        ],
    )(_sc_layer_body)
    return f(y, gidx_p, bw_p, dst_p)


def _bn(pre, g, be):
    mu = jnp.mean(pre, axis=0)
    d = pre - mu
    var = jnp.mean(d * d, axis=0)
    return g * d / jnp.sqrt(var + 1e-5) + be


def _elu(v):
    return jnp.where(v > 0.0, v, jnp.exp(jnp.minimum(v, 0.0)) - 1.0)


def _cnt_full(cnt_ref):
    return jnp.concatenate([cnt_ref[0], cnt_ref[1]], axis=0)[0:N, :]


def _post1_body(agg_ref, cnt_ref, x_ref, root_ref, b_ref, g_ref, be_ref,
                h_ref):
    cnt = _cnt_full(cnt_ref)
    agg = (agg_ref[0][0:N, :] + agg_ref[1][0:N, :]) / jnp.maximum(cnt, 1.0)
    pre = agg + jnp.dot(x_ref[...], root_ref[...],
                        preferred_element_type=jnp.float32) + b_ref[...]
    h_ref[...] = _elu(_bn(pre, g_ref[...], be_ref[...]))


def _post1(agg, cnt, x, root, b, g, be):
    return pl.pallas_call(
        _post1_body,
        out_shape=jax.ShapeDtypeStruct((N, C), jnp.float32),
    )(agg, cnt, x, root, b, g, be)


def _post2_body(agg_ref, aggx_ref, cnt_ref, h_ref, x_ref,
                root2_ref, b2_ref, g2_ref, be2_ref,
                ws_ref, roots_ref, bs_ref, gs_ref, bes_ref, out_ref):
    cnt = jnp.maximum(_cnt_full(cnt_ref), 1.0)
    agg = (agg_ref[0][0:N, :] + agg_ref[1][0:N, :]) / cnt
    left_pre = agg + jnp.dot(h_ref[...], root2_ref[...],
                             preferred_element_type=jnp.float32) + b2_ref[...]
    left = _bn(left_pre, g2_ref[...], be2_ref[...])
    aggx = jnp.concatenate([aggx_ref[0], aggx_ref[1]], axis=0)[0:N, :] / cnt
    short_pre = (jnp.dot(aggx, ws_ref[...], preferred_element_type=jnp.float32)
                 + jnp.dot(x_ref[...], roots_ref[...],
                           preferred_element_type=jnp.float32) + bs_ref[...])
    short = _bn(short_pre, gs_ref[...], bes_ref[...])
    out_ref[...] = _elu(left + short)


def _post2(agg, aggx, cnt, h, x, root2, b2, g2, be2, ws, roots, bs, gs, bes):
    return pl.pallas_call(
        _post2_body,
        out_shape=jax.ShapeDtypeStruct((N, C), jnp.float32),
    )(agg, aggx, cnt, h, x, root2, b2, g2, be2, ws, roots, bs, gs, bes)


def kernel(x, edge_index, edge_attr, W1, root1, b1, g1, be1,
           W2, root2, b2, g2, be2, Ws, roots, bs, gs, bes):
    edge_index = edge_index.astype(jnp.int32)
    src = edge_index[0]
    dst = edge_index[1]
    attr_t = edge_attr.T

    gidx, bw = _basis(edge_index, attr_t)
    pad = ((0, 0), (0, EPTP - EPT), (0, 0))
    gidx_p = jnp.pad(gidx.T.reshape(NW, EPT, 8), pad).reshape(NW * NCHT * 128)
    bw_p = jnp.pad(bw.T.reshape(NW, EPT, 8), pad).reshape(NW * NCHT * 128)
    dst_p = jnp.pad(dst.reshape(NW, EPT), ((0, 0), (0, EPTP - EPT))
                    ).reshape(NW * NCHT * 16)

    y1 = _ymm(x, W1)
    aggx, cnt = _sc_short(x, src, dst)
    agg1 = _sc_layer(y1, gidx_p, bw_p, dst_p)
    h = _post1(agg1, cnt, x, root1, b1, g1, be1)
    y2 = _ymm(h, W2)
    agg2 = _sc_layer(y2, gidx_p, bw_p, dst_p)
    return _post2(agg2, aggx, cnt, h, x, root2, b2, g2, be2,
                  Ws[0], roots, bs, gs, bes)


# speedup vs baseline: 1.0787x; 1.0136x over previous
import functools

import jax
import jax.numpy as jnp
from jax import lax
from jax.experimental import pallas as pl
from jax.experimental.pallas import tpu as pltpu
from jax.experimental.pallas import tpu_sc as plsc

N = 10000
E = 160000
C = 128
K5 = 125

NC = 2
NS = 16
NW = NC * NS
NP = 10240
NPT = NP // NS
ZB = 16

CBL = 16
EPT = E // NW
NCHT = 316
EPTP = NCHT * CBL
NQUAD = NCHT // 4

PIPELINED = True

CBS = 64
NCH_S = E // CBS


def _basis_body(ei_ref, at_ref, gidx_ref, bw_ref):
    src = ei_ref[0, :]
    p0 = at_ref[0, :] * 4.0
    p1 = at_ref[1, :] * 4.0
    p2 = at_ref[2, :] * 4.0
    fl0 = jnp.floor(p0)
    fl1 = jnp.floor(p1)
    fl2 = jnp.floor(p2)
    fr = (p0 - fl0, p1 - fl1, p2 - fl2)
    fli = (fl0.astype(jnp.int32), fl1.astype(jnp.int32), fl2.astype(jnp.int32))
    off = (1, 5, 25)
    for s in range(8):
        b = None
        wi = None
        for d in range(3):
            bit = (s >> d) & 1
            v = fr[d] if bit == 1 else (1.0 - fr[d])
            b = v if b is None else b * v
            idx = (fli[d] + bit) % 5
            t = idx * off[d]
            wi = t if wi is None else wi + t
        gidx_ref[s, :] = wi * N + src
        bw_ref[s, :] = b


def _basis(edge_index, attr_t):
    return pl.pallas_call(
        _basis_body,
        out_shape=(
            jax.ShapeDtypeStruct((8, E), jnp.int32),
            jax.ShapeDtypeStruct((8, E), jnp.float32),
        ),
    )(edge_index, attr_t)


BN_N = 1000


def _ymm_body(x_ref, w_ref, y_ref):
    y_ref[0] = jnp.dot(x_ref[...], w_ref[0],
                       preferred_element_type=jnp.float32)


def _ymm(x, w):
    nt = N // BN_N
    y = pl.pallas_call(
        _ymm_body,
        grid=(nt, K5),
        in_specs=[
            pl.BlockSpec((BN_N, C), lambda n, k: (n, 0)),
            pl.BlockSpec((1, C, C), lambda n, k: (k, 0, 0)),
        ],
        out_specs=pl.BlockSpec((1, BN_N, C), lambda n, k: (k, n, 0)),
        out_shape=jax.ShapeDtypeStruct((K5, N, C), jnp.float32),
    )(x.astype(jnp.bfloat16), w.astype(jnp.bfloat16))
    return y.reshape(K5 * N, C)


def _zero_fill(zb_v, width):
    def _zrow(r, _):
        for cc in range(width // 16):
            zb_v[r, pl.ds(cc * 16, 16)] = jnp.zeros((16,), jnp.float32)
        return 0

    lax.fori_loop(0, ZB, _zrow, 0)


def _zero_spmem(zb_v, sh, s, width):
    for r in range(NPT // ZB):
        pltpu.sync_copy(zb_v.at[0:ZB, 0:width],
                        sh.at[pl.ds(s * NPT + r * ZB, ZB)])


HP = NP // NC
HPT = HP // NS


def _sc_short_body(x_hbm, src_hbm, dst_hbm, aggx_hbm, cnt_hbm,
                   src_v, dst_v, ldst_v, rows_v, ones_v, zb_v,
                   aggx_sh, cnt_sh, sem):
    c = lax.axis_index("c")
    s = lax.axis_index("s")

    _zero_fill(zb_v, C)

    def _orow(r, _):
        for cc in range(8):
            ones_v[r, pl.ds(cc * 16, 16)] = jnp.ones((16,), jnp.float32)
        return 0

    lax.fori_loop(0, CBS, _orow, 0)

    for r in range(HPT // ZB):
        pltpu.sync_copy(zb_v.at[0:ZB, :],
                        aggx_sh.at[pl.ds(s * HPT + r * ZB, ZB)])
        pltpu.sync_copy(zb_v.at[0:ZB, :],
                        cnt_sh.at[pl.ds(s * HPT + r * ZB, ZB)])

    @pl.when(s == 0)
    def _():
        pltpu.sync_copy(zb_v.at[0:16, :], aggx_sh.at[pl.ds(HP, 16)])
        pltpu.sync_copy(zb_v.at[0:16, :], cnt_sh.at[pl.ds(HP, 16)])

    plsc.subcore_barrier()
    lo = c * HP

    def _chunk(j, _):
        cid = s + j * NS

        @pl.when(cid < NCH_S)
        def _():
            base = cid * CBS
            pltpu.sync_copy(src_hbm.at[pl.ds(base, CBS)], src_v)
            pltpu.sync_copy(dst_hbm.at[pl.ds(base, CBS)], dst_v)
            for r in range(CBS // 16):
                sl = pl.ds(r * 16, 16)
                d = dst_v[sl] - lo
                ok = (d >= 0) & (d < HP)
                ldst_v[sl] = jnp.where(ok, d, HP)
            pltpu.async_copy(x_hbm.at[src_v], rows_v, sem).wait()
            pltpu.sync_copy(rows_v, aggx_sh.at[ldst_v], add=True)
            pltpu.sync_copy(ones_v, cnt_sh.at[ldst_v], add=True)

        return 0

    lax.fori_loop(0, NIT_S2, _chunk, 0)
    plsc.subcore_barrier()

    pltpu.sync_copy(aggx_sh.at[pl.ds(s * HPT, HPT)],
                    aggx_hbm.at[c, pl.ds(s * HPT, HPT)])
    pltpu.sync_copy(cnt_sh.at[pl.ds(s * HPT, HPT)],
                    cnt_hbm.at[c, pl.ds(s * HPT, HPT)])


NIT_S2 = (NCH_S + NS - 1) // NS


def _sc_short(x, src, dst):
    mesh = plsc.VectorSubcoreMesh(core_axis_name="c", subcore_axis_name="s")
    f = functools.partial(
        pl.kernel,
        mesh=mesh,
        out_type=(
            jax.ShapeDtypeStruct((NC, HP, C), jnp.float32),
            jax.ShapeDtypeStruct((NC, HP, C), jnp.float32),
        ),
        scratch_types=[
            pltpu.VMEM((CBS,), jnp.int32),
            pltpu.VMEM((CBS,), jnp.int32),
            pltpu.VMEM((CBS,), jnp.int32),
            pltpu.VMEM((CBS, C), jnp.float32),
            pltpu.VMEM((CBS, C), jnp.float32),
            pltpu.VMEM((ZB, C), jnp.float32),
            pltpu.VMEM_SHARED((HP + 16, C), jnp.float32),
            pltpu.VMEM_SHARED((HP + 16, C), jnp.float32),
            pltpu.SemaphoreType.DMA,
        ],
    )(_sc_short_body)
    return f(x, src, dst)


def _sc_layer_body(y_hbm, gidx_hbm, bw_hbm, dst_hbm, agg_hbm,
                   gidx_v, bw_v, dstm_v, rows_v, msg_v, zb_v, agg_sh,
                   semg0, semg1, semm0, semm1, sems0, sems1):
    c = lax.axis_index("c")
    s = lax.axis_index("s")
    wid = s * NC + c

    _zero_fill(zb_v, C)
    _zero_spmem(zb_v, agg_sh, s, C)
    plsc.subcore_barrier()

    gb = wid * (NCHT * 128)
    db = wid * (NCHT * 16)
    semg = (semg0, semg1)
    semm = (semm0, semm1)
    sems_s = (sems0, sems1)

    def _meta_copies(m, j0):
        return [
            pltpu.make_async_copy(gidx_hbm.at[pl.ds(gb + j0 * 128, 256)],
                                  gidx_v.at[m], semm[m]),
            pltpu.make_async_copy(bw_hbm.at[pl.ds(gb + j0 * 128, 128)],
                                  bw_v.at[m, 0], semm[m]),
            pltpu.make_async_copy(bw_hbm.at[pl.ds(gb + j0 * 128 + 128, 128)],
                                  bw_v.at[m, 1], semm[m]),
            pltpu.make_async_copy(dst_hbm.at[pl.ds(db + j0 * 16, 16)],
                                  dstm_v.at[m, 0], semm[m]),
            pltpu.make_async_copy(dst_hbm.at[pl.ds(db + j0 * 16 + 16, 16)],
                                  dstm_v.at[m, 1], semm[m]),
        ]

    def _meta_start(m, j0):
        for cp in _meta_copies(m, j0):
            cp.start()

    def _meta_wait(m):
        for cp in _meta_copies(m, 0):
            cp.wait()

    NSPL = 4

    def _gather_copies(m, jj, p):
        w = 128 // NSPL
        return [
            pltpu.make_async_copy(
                y_hbm.at[gidx_v.at[m, pl.ds(jj * 128 + i * w, w)]],
                rows_v.at[p, pl.ds(i * w, w)], semg[p])
            for i in range(NSPL)
        ]

    def _gather_start(m, jj, p):
        for cp in _gather_copies(m, jj, p):
            cp.start()

    def _gather_wait(m, jj, p):
        for cp in _gather_copies(m, jj, p):
            cp.wait()

    def _scatter_wait(p):
        pltpu.make_async_copy(msg_v.at[p], agg_sh.at[dstm_v.at[0, 0]],
                              sems_s[p]).wait()

    def _combine(m, jj, p):
        _scatter_wait(p)

        def _epair(e2, _):
            v = bw_v[m, jj, pl.ds(e2 * 16, 16)]
            for half in range(2):
                eb = e2 * 16 + half * 8
                for cc in range(8):
                    sl = pl.ds(cc * 16, 16)
                    acc = v[half * 8] * rows_v[p, eb, sl]
                    for t in range(1, 8):
                        acc = acc + v[half * 8 + t] * rows_v[p, eb + t, sl]
                    msg_v[p, e2 * 2 + half, sl] = acc
            return 0

        lax.fori_loop(0, CBL // 2, _epair, 0)
        pltpu.async_copy(msg_v.at[p], agg_sh.at[dstm_v.at[m, jj]],
                         sems_s[p], add=True)

    def _quad(q, _):
        nx0 = lax.rem(4 * q + 4, NCHT)
        nx1 = lax.rem(4 * q + 6, NCHT)
        _gather_wait(0, 0, 0)
        _combine(0, 0, 0)
        _gather_start(1, 0, 0)
        _gather_wait(0, 1, 1)
        _combine(0, 1, 1)
        _meta_start(0, nx0)
        _gather_start(1, 1, 1)
        _gather_wait(1, 0, 0)
        _combine(1, 0, 0)
        _meta_wait(0)
        _gather_start(0, 0, 0)
        _gather_wait(1, 1, 1)
        _combine(1, 1, 1)
        _meta_start(1, nx1)
        _meta_wait(1)
        _gather_start(0, 1, 1)
        return 0

    def _quad_serial(q, _):
        nx0 = lax.rem(4 * q + 4, NCHT)
        nx1 = lax.rem(4 * q + 6, NCHT)
        for m, jj in ((0, 0), (0, 1), (1, 0), (1, 1)):
            _gather_start(m, jj, 0)
            _gather_wait(m, jj, 0)
            _combine(m, jj, 0)
        _meta_start(0, nx0)
        _meta_wait(0)
        _meta_start(1, nx1)
        _meta_wait(1)
        return 0

    _meta_start(0, 0)
    _meta_start(1, 2)
    _meta_wait(0)
    _meta_wait(1)
    pltpu.async_copy(zb_v, agg_sh.at[dstm_v.at[0, 0]], sems_s[0], add=True)
    pltpu.async_copy(zb_v, agg_sh.at[dstm_v.at[0, 0]], sems_s[1], add=True)
    if PIPELINED:
        _gather_start(0, 0, 0)
        _gather_start(0, 1, 1)
        lax.fori_loop(0, NQUAD, _quad, 0)
        _gather_wait(0, 0, 0)
        _gather_wait(0, 1, 1)
    else:
        lax.fori_loop(0, NQUAD, _quad_serial, 0)
    _scatter_wait(0)
    _scatter_wait(1)

    plsc.subcore_barrier()
    pltpu.sync_copy(agg_sh.at[pl.ds(s * NPT, NPT)],
                    agg_hbm.at[c, pl.ds(s * NPT, NPT)])


def _sc_layer(y, gidx_p, bw_p, dst_p):
    mesh = plsc.VectorSubcoreMesh(core_axis_name="c", subcore_axis_name="s")
    f = functools.partial(
        pl.kernel,
        mesh=mesh,
        out_type=jax.ShapeDtypeStruct((NC, NP, C), jnp.float32),
        scratch_types=[
            pltpu.VMEM((2, 256), jnp.int32),
            pltpu.VMEM((2, 2, 128), jnp.float32),
            pltpu.VMEM((2, 2, 16), jnp.int32),
            pltpu.VMEM((2, 128, C), jnp.float32),
            pltpu.VMEM((2, CBL, C), jnp.float32),
            pltpu.VMEM((ZB, C), jnp.float32),
            pltpu.VMEM_SHARED((NP, C), jnp.float32),
            pltpu.SemaphoreType.DMA,
            pltpu.SemaphoreType.DMA,
            pltpu.SemaphoreType.DMA,
            pltpu.SemaphoreType.DMA,
            pltpu.SemaphoreType.DMA,
            pltpu.SemaphoreType.DMA,
        ],
    )(_sc_layer_body)
    return f(y, gidx_p, bw_p, dst_p)


def _bn(pre, g, be):
    mu = jnp.mean(pre, axis=0)
    d = pre - mu
    var = jnp.mean(d * d, axis=0)
    return g * d / jnp.sqrt(var + 1e-5) + be


def _elu(v):
    return jnp.where(v > 0.0, v, jnp.exp(jnp.minimum(v, 0.0)) - 1.0)


def _cnt_full(cnt_ref):
    return jnp.concatenate([cnt_ref[0], cnt_ref[1]], axis=0)[0:N, :]


def _post1_body(agg_ref, cnt_ref, x_ref, root_ref, b_ref, g_ref, be_ref,
                h_ref):
    cnt = _cnt_full(cnt_ref)
    agg = (agg_ref[0][0:N, :] + agg_ref[1][0:N, :]) / jnp.maximum(cnt, 1.0)
    pre = agg + jnp.dot(x_ref[...], root_ref[...],
                        preferred_element_type=jnp.float32) + b_ref[...]
    h_ref[...] = _elu(_bn(pre, g_ref[...], be_ref[...]))


def _post1(agg, cnt, x, root, b, g, be):
    return pl.pallas_call(
        _post1_body,
        out_shape=jax.ShapeDtypeStruct((N, C), jnp.float32),
    )(agg, cnt, x, root, b, g, be)


def _post2_body(agg_ref, aggx_ref, cnt_ref, h_ref, x_ref,
                root2_ref, b2_ref, g2_ref, be2_ref,
                ws_ref, roots_ref, bs_ref, gs_ref, bes_ref, out_ref):
    cnt = jnp.maximum(_cnt_full(cnt_ref), 1.0)
    agg = (agg_ref[0][0:N, :] + agg_ref[1][0:N, :]) / cnt
    left_pre = agg + jnp.dot(h_ref[...], root2_ref[...],
                             preferred_element_type=jnp.float32) + b2_ref[...]
    left = _bn(left_pre, g2_ref[...], be2_ref[...])
    aggx = jnp.concatenate([aggx_ref[0], aggx_ref[1]], axis=0)[0:N, :] / cnt
    short_pre = (jnp.dot(aggx, ws_ref[...], preferred_element_type=jnp.float32)
                 + jnp.dot(x_ref[...], roots_ref[...],
                           preferred_element_type=jnp.float32) + bs_ref[...])
    short = _bn(short_pre, gs_ref[...], bes_ref[...])
    out_ref[...] = _elu(left + short)


def _post2(agg, aggx, cnt, h, x, root2, b2, g2, be2, ws, roots, bs, gs, bes):
    return pl.pallas_call(
        _post2_body,
        out_shape=jax.ShapeDtypeStruct((N, C), jnp.float32),
    )(agg, aggx, cnt, h, x, root2, b2, g2, be2, ws, roots, bs, gs, bes)


def kernel(x, edge_index, edge_attr, W1, root1, b1, g1, be1,
           W2, root2, b2, g2, be2, Ws, roots, bs, gs, bes):
    edge_index = edge_index.astype(jnp.int32)
    src = edge_index[0]
    dst = edge_index[1]
    attr_t = edge_attr.T

    gidx, bw = _basis(edge_index, attr_t)
    pad = ((0, 0), (0, EPTP - EPT), (0, 0))
    gidx_p = jnp.pad(gidx.T.reshape(NW, EPT, 8), pad).reshape(NW * NCHT * 128)
    bw_p = jnp.pad(bw.T.reshape(NW, EPT, 8), pad).reshape(NW * NCHT * 128)
    dst_p = jnp.pad(dst.reshape(NW, EPT), ((0, 0), (0, EPTP - EPT))
                    ).reshape(NW * NCHT * 16)

    y1 = _ymm(x, W1)
    aggx, cnt = _sc_short(x, src, dst)
    agg1 = _sc_layer(y1, gidx_p, bw_p, dst_p)
    h = _post1(agg1, cnt, x, root1, b1, g1, be1)
    y2 = _ymm(h, W2)
    agg2 = _sc_layer(y2, gidx_p, bw_p, dst_p)
    return _post2(agg2, aggx, cnt, h, x, root2, b2, g2, be2,
                  Ws[0], roots, bs, gs, bes)


# speedup vs baseline: 1.1305x; 1.0480x over previous
import functools

import jax
import jax.numpy as jnp
from jax import lax
from jax.experimental import pallas as pl
from jax.experimental.pallas import tpu as pltpu
from jax.experimental.pallas import tpu_sc as plsc

N = 10000
E = 160000
C = 128
K5 = 125

NC = 2
NS = 16
NW = NC * NS
NP = 10240
NPT = NP // NS
ZB = 16

CBL = 16
EPT = E // NW
NCHT = 316
EPTP = NCHT * CBL
NQUAD = NCHT // 4

PIPELINED = True

CBS = 64
NCH_S = E // CBS


def _basis_body(ei_ref, at_ref, gidx_ref, bw_ref):
    src = ei_ref[0, :]
    p0 = at_ref[0, :] * 4.0
    p1 = at_ref[1, :] * 4.0
    p2 = at_ref[2, :] * 4.0
    fl0 = jnp.floor(p0)
    fl1 = jnp.floor(p1)
    fl2 = jnp.floor(p2)
    fr = (p0 - fl0, p1 - fl1, p2 - fl2)
    fli = (fl0.astype(jnp.int32), fl1.astype(jnp.int32), fl2.astype(jnp.int32))
    off = (1, 5, 25)
    for s in range(8):
        b = None
        wi = None
        for d in range(3):
            bit = (s >> d) & 1
            v = fr[d] if bit == 1 else (1.0 - fr[d])
            b = v if b is None else b * v
            idx = (fli[d] + bit) % 5
            t = idx * off[d]
            wi = t if wi is None else wi + t
        gidx_ref[s, :] = wi * N + src
        bw_ref[s, :] = b


def _basis(edge_index, attr_t):
    return pl.pallas_call(
        _basis_body,
        out_shape=(
            jax.ShapeDtypeStruct((8, E), jnp.int32),
            jax.ShapeDtypeStruct((8, E), jnp.float32),
        ),
    )(edge_index, attr_t)


BN_N = 1000


def _ymm_body(x_ref, w_ref, y_ref):
    y_ref[0] = jnp.dot(x_ref[...], w_ref[0],
                       preferred_element_type=jnp.float32)


def _ymm(x, w):
    nt = N // BN_N
    y = pl.pallas_call(
        _ymm_body,
        grid=(nt, K5),
        in_specs=[
            pl.BlockSpec((BN_N, C), lambda n, k: (n, 0)),
            pl.BlockSpec((1, C, C), lambda n, k: (k, 0, 0)),
        ],
        out_specs=pl.BlockSpec((1, BN_N, C), lambda n, k: (k, n, 0)),
        out_shape=jax.ShapeDtypeStruct((K5, N, C), jnp.float32),
    )(x.astype(jnp.bfloat16), w.astype(jnp.bfloat16))
    return y.reshape(K5 * N, C)


def _zero_fill(zb_v, width):
    def _zrow(r, _):
        for cc in range(width // 16):
            zb_v[r, pl.ds(cc * 16, 16)] = jnp.zeros((16,), jnp.float32)
        return 0

    lax.fori_loop(0, ZB, _zrow, 0)


def _zero_spmem(zb_v, sh, s, width):
    for r in range(NPT // ZB):
        pltpu.sync_copy(zb_v.at[0:ZB, 0:width],
                        sh.at[pl.ds(s * NPT + r * ZB, ZB)])


HP = NP // NC
HPT = HP // NS


def _sc_short_body(x_hbm, src_hbm, dst_hbm, aggx_hbm, cnt_hbm,
                   src_v, dst_v, ldst_v, rows_v, ones_v, zb_v,
                   aggx_sh, cnt_sh, sem):
    c = lax.axis_index("c")
    s = lax.axis_index("s")

    _zero_fill(zb_v, C)

    def _orow(r, _):
        for cc in range(8):
            ones_v[r, pl.ds(cc * 16, 16)] = jnp.ones((16,), jnp.float32)
        return 0

    lax.fori_loop(0, CBS, _orow, 0)

    for r in range(HPT // ZB):
        pltpu.sync_copy(zb_v.at[0:ZB, :],
                        aggx_sh.at[pl.ds(s * HPT + r * ZB, ZB)])
        pltpu.sync_copy(zb_v.at[0:ZB, :],
                        cnt_sh.at[pl.ds(s * HPT + r * ZB, ZB)])

    @pl.when(s == 0)
    def _():
        pltpu.sync_copy(zb_v.at[0:16, :], aggx_sh.at[pl.ds(HP, 16)])
        pltpu.sync_copy(zb_v.at[0:16, :], cnt_sh.at[pl.ds(HP, 16)])

    plsc.subcore_barrier()
    lo = c * HP

    def _chunk(j, _):
        cid = s + j * NS

        @pl.when(cid < NCH_S)
        def _():
            base = cid * CBS
            pltpu.sync_copy(src_hbm.at[pl.ds(base, CBS)], src_v)
            pltpu.sync_copy(dst_hbm.at[pl.ds(base, CBS)], dst_v)
            for r in range(CBS // 16):
                sl = pl.ds(r * 16, 16)
                d = dst_v[sl] - lo
                ok = (d >= 0) & (d < HP)
                ldst_v[sl] = jnp.where(ok, d, HP)
            pltpu.async_copy(x_hbm.at[src_v], rows_v, sem).wait()
            pltpu.sync_copy(rows_v, aggx_sh.at[ldst_v], add=True)
            pltpu.sync_copy(ones_v, cnt_sh.at[ldst_v], add=True)

        return 0

    lax.fori_loop(0, NIT_S2, _chunk, 0)
    plsc.subcore_barrier()

    pltpu.sync_copy(aggx_sh.at[pl.ds(s * HPT, HPT)],
                    aggx_hbm.at[c, pl.ds(s * HPT, HPT)])
    pltpu.sync_copy(cnt_sh.at[pl.ds(s * HPT, HPT)],
                    cnt_hbm.at[c, pl.ds(s * HPT, HPT)])


NIT_S2 = (NCH_S + NS - 1) // NS


def _sc_short(x, src, dst):
    mesh = plsc.VectorSubcoreMesh(core_axis_name="c", subcore_axis_name="s")
    f = functools.partial(
        pl.kernel,
        mesh=mesh,
        out_type=(
            jax.ShapeDtypeStruct((NC, HP, C), jnp.float32),
            jax.ShapeDtypeStruct((NC, HP, C), jnp.float32),
        ),
        scratch_types=[
            pltpu.VMEM((CBS,), jnp.int32),
            pltpu.VMEM((CBS,), jnp.int32),
            pltpu.VMEM((CBS,), jnp.int32),
            pltpu.VMEM((CBS, C), jnp.float32),
            pltpu.VMEM((CBS, C), jnp.float32),
            pltpu.VMEM((ZB, C), jnp.float32),
            pltpu.VMEM_SHARED((HP + 16, C), jnp.float32),
            pltpu.VMEM_SHARED((HP + 16, C), jnp.float32),
            pltpu.SemaphoreType.DMA,
        ],
    )(_sc_short_body)
    return f(x, src, dst)


def _sc_layer_body(y_hbm, gidx_hbm, bw_hbm, dst_hbm, agg_hbm,
                   gidx_v, bw_v, dstm_v, rows_v, msg_v, zb_v, agg_sh,
                   semg0, semg1, semm0, semm1, sems0, sems1):
    c = lax.axis_index("c")
    s = lax.axis_index("s")
    wid = s * NC + c

    _zero_fill(zb_v, C)
    _zero_spmem(zb_v, agg_sh, s, C)
    plsc.subcore_barrier()

    gb = wid * (NCHT * 128)
    db = wid * (NCHT * 16)
    semg = (semg0, semg1)
    semm = (semm0, semm1)
    sems_s = (sems0, sems1)

    def _meta_copies(m, j0):
        return [
            pltpu.make_async_copy(gidx_hbm.at[pl.ds(gb + j0 * 128, 256)],
                                  gidx_v.at[m], semm[m]),
            pltpu.make_async_copy(bw_hbm.at[pl.ds(gb + j0 * 128, 128)],
                                  bw_v.at[m, 0], semm[m]),
            pltpu.make_async_copy(bw_hbm.at[pl.ds(gb + j0 * 128 + 128, 128)],
                                  bw_v.at[m, 1], semm[m]),
            pltpu.make_async_copy(dst_hbm.at[pl.ds(db + j0 * 16, 16)],
                                  dstm_v.at[m, 0], semm[m]),
            pltpu.make_async_copy(dst_hbm.at[pl.ds(db + j0 * 16 + 16, 16)],
                                  dstm_v.at[m, 1], semm[m]),
        ]

    def _meta_start(m, j0):
        for cp in _meta_copies(m, j0):
            cp.start()

    def _meta_wait(m):
        for cp in _meta_copies(m, 0):
            cp.wait()

    NSPL = 4

    def _gather_copies(m, jj, p):
        w = 128 // NSPL
        return [
            pltpu.make_async_copy(
                y_hbm.at[gidx_v.at[m, pl.ds(jj * 128 + i * w, w)]],
                rows_v.at[p, pl.ds(i * w, w)], semg[p])
            for i in range(NSPL)
        ]

    def _gather_start(m, jj, p):
        for cp in _gather_copies(m, jj, p):
            cp.start()

    def _gather_wait(m, jj, p):
        for cp in _gather_copies(m, jj, p):
            cp.wait()

    def _scatter_wait(p):
        pltpu.make_async_copy(msg_v.at[p], agg_sh.at[dstm_v.at[0, 0]],
                              sems_s[p]).wait()

    def _combine(m, jj, p):
        _scatter_wait(p)

        def _epair(e2, _):
            v = bw_v[m, jj, pl.ds(e2 * 16, 16)]
            for half in range(2):
                eb = e2 * 16 + half * 8
                b = [v[half * 8 + t] for t in range(8)]
                for cc in range(8):
                    sl = pl.ds(cc * 16, 16)
                    pr = [b[t] * rows_v[p, eb + t, sl] for t in range(8)]
                    acc = (((pr[0] + pr[1]) + (pr[2] + pr[3]))
                           + ((pr[4] + pr[5]) + (pr[6] + pr[7])))
                    msg_v[p, e2 * 2 + half, sl] = acc
            return 0

        lax.fori_loop(0, CBL // 2, _epair, 0)
        pltpu.async_copy(msg_v.at[p], agg_sh.at[dstm_v.at[m, jj]],
                         sems_s[p], add=True)

    def _quad(q, _):
        nx0 = lax.rem(4 * q + 4, NCHT)
        nx1 = lax.rem(4 * q + 6, NCHT)
        _gather_wait(0, 0, 0)
        _combine(0, 0, 0)
        _gather_start(1, 0, 0)
        _gather_wait(0, 1, 1)
        _combine(0, 1, 1)
        _meta_start(0, nx0)
        _gather_start(1, 1, 1)
        _gather_wait(1, 0, 0)
        _combine(1, 0, 0)
        _meta_wait(0)
        _gather_start(0, 0, 0)
        _gather_wait(1, 1, 1)
        _combine(1, 1, 1)
        _meta_start(1, nx1)
        _meta_wait(1)
        _gather_start(0, 1, 1)
        return 0

    def _quad_serial(q, _):
        nx0 = lax.rem(4 * q + 4, NCHT)
        nx1 = lax.rem(4 * q + 6, NCHT)
        for m, jj in ((0, 0), (0, 1), (1, 0), (1, 1)):
            _gather_start(m, jj, 0)
            _gather_wait(m, jj, 0)
            _combine(m, jj, 0)
        _meta_start(0, nx0)
        _meta_wait(0)
        _meta_start(1, nx1)
        _meta_wait(1)
        return 0

    _meta_start(0, 0)
    _meta_start(1, 2)
    _meta_wait(0)
    _meta_wait(1)
    pltpu.async_copy(zb_v, agg_sh.at[dstm_v.at[0, 0]], sems_s[0], add=True)
    pltpu.async_copy(zb_v, agg_sh.at[dstm_v.at[0, 0]], sems_s[1], add=True)
    if PIPELINED:
        _gather_start(0, 0, 0)
        _gather_start(0, 1, 1)
        lax.fori_loop(0, NQUAD, _quad, 0)
        _gather_wait(0, 0, 0)
        _gather_wait(0, 1, 1)
    else:
        lax.fori_loop(0, NQUAD, _quad_serial, 0)
    _scatter_wait(0)
    _scatter_wait(1)

    plsc.subcore_barrier()
    pltpu.sync_copy(agg_sh.at[pl.ds(s * NPT, NPT)],
                    agg_hbm.at[c, pl.ds(s * NPT, NPT)])


def _sc_layer(y, gidx_p, bw_p, dst_p):
    mesh = plsc.VectorSubcoreMesh(core_axis_name="c", subcore_axis_name="s")
    f = functools.partial(
        pl.kernel,
        mesh=mesh,
        out_type=jax.ShapeDtypeStruct((NC, NP, C), jnp.float32),
        scratch_types=[
            pltpu.VMEM((2, 256), jnp.int32),
            pltpu.VMEM((2, 2, 128), jnp.float32),
            pltpu.VMEM((2, 2, 16), jnp.int32),
            pltpu.VMEM((2, 128, C), jnp.float32),
            pltpu.VMEM((2, CBL, C), jnp.float32),
            pltpu.VMEM((ZB, C), jnp.float32),
            pltpu.VMEM_SHARED((NP, C), jnp.float32),
            pltpu.SemaphoreType.DMA,
            pltpu.SemaphoreType.DMA,
            pltpu.SemaphoreType.DMA,
            pltpu.SemaphoreType.DMA,
            pltpu.SemaphoreType.DMA,
            pltpu.SemaphoreType.DMA,
        ],
    )(_sc_layer_body)
    return f(y, gidx_p, bw_p, dst_p)


def _bn(pre, g, be):
    mu = jnp.mean(pre, axis=0)
    d = pre - mu
    var = jnp.mean(d * d, axis=0)
    return g * d / jnp.sqrt(var + 1e-5) + be


def _elu(v):
    return jnp.where(v > 0.0, v, jnp.exp(jnp.minimum(v, 0.0)) - 1.0)


def _cnt_full(cnt_ref):
    return jnp.concatenate([cnt_ref[0], cnt_ref[1]], axis=0)[0:N, :]


def _post1_body(agg_ref, cnt_ref, x_ref, root_ref, b_ref, g_ref, be_ref,
                h_ref):
    cnt = _cnt_full(cnt_ref)
    agg = (agg_ref[0][0:N, :] + agg_ref[1][0:N, :]) / jnp.maximum(cnt, 1.0)
    pre = agg + jnp.dot(x_ref[...], root_ref[...],
                        preferred_element_type=jnp.float32) + b_ref[...]
    h_ref[...] = _elu(_bn(pre, g_ref[...], be_ref[...]))


def _post1(agg, cnt, x, root, b, g, be):
    return pl.pallas_call(
        _post1_body,
        out_shape=jax.ShapeDtypeStruct((N, C), jnp.float32),
    )(agg, cnt, x, root, b, g, be)


def _post2_body(agg_ref, aggx_ref, cnt_ref, h_ref, x_ref,
                root2_ref, b2_ref, g2_ref, be2_ref,
                ws_ref, roots_ref, bs_ref, gs_ref, bes_ref, out_ref):
    cnt = jnp.maximum(_cnt_full(cnt_ref), 1.0)
    agg = (agg_ref[0][0:N, :] + agg_ref[1][0:N, :]) / cnt
    left_pre = agg + jnp.dot(h_ref[...], root2_ref[...],
                             preferred_element_type=jnp.float32) + b2_ref[...]
    left = _bn(left_pre, g2_ref[...], be2_ref[...])
    aggx = jnp.concatenate([aggx_ref[0], aggx_ref[1]], axis=0)[0:N, :] / cnt
    short_pre = (jnp.dot(aggx, ws_ref[...], preferred_element_type=jnp.float32)
                 + jnp.dot(x_ref[...], roots_ref[...],
                           preferred_element_type=jnp.float32) + bs_ref[...])
    short = _bn(short_pre, gs_ref[...], bes_ref[...])
    out_ref[...] = _elu(left + short)


def _post2(agg, aggx, cnt, h, x, root2, b2, g2, be2, ws, roots, bs, gs, bes):
    return pl.pallas_call(
        _post2_body,
        out_shape=jax.ShapeDtypeStruct((N, C), jnp.float32),
    )(agg, aggx, cnt, h, x, root2, b2, g2, be2, ws, roots, bs, gs, bes)


def kernel(x, edge_index, edge_attr, W1, root1, b1, g1, be1,
           W2, root2, b2, g2, be2, Ws, roots, bs, gs, bes):
    edge_index = edge_index.astype(jnp.int32)
    src = edge_index[0]
    dst = edge_index[1]
    attr_t = edge_attr.T

    gidx, bw = _basis(edge_index, attr_t)
    pad = ((0, 0), (0, EPTP - EPT), (0, 0))
    gidx_p = jnp.pad(gidx.T.reshape(NW, EPT, 8), pad).reshape(NW * NCHT * 128)
    bw_p = jnp.pad(bw.T.reshape(NW, EPT, 8), pad).reshape(NW * NCHT * 128)
    dst_p = jnp.pad(dst.reshape(NW, EPT), ((0, 0), (0, EPTP - EPT))
                    ).reshape(NW * NCHT * 16)

    y1 = _ymm(x, W1)
    aggx, cnt = _sc_short(x, src, dst)
    agg1 = _sc_layer(y1, gidx_p, bw_p, dst_p)
    h = _post1(agg1, cnt, x, root1, b1, g1, be1)
    y2 = _ymm(h, W2)
    agg2 = _sc_layer(y2, gidx_p, bw_p, dst_p)
    return _post2(agg2, aggx, cnt, h, x, root2, b2, g2, be2,
                  Ws[0], roots, bs, gs, bes)


# speedup vs baseline: 1.3410x; 1.1862x over previous
import functools

import jax
import jax.numpy as jnp
from jax import lax
from jax.experimental import pallas as pl
from jax.experimental.pallas import tpu as pltpu
from jax.experimental.pallas import tpu_sc as plsc

N = 10000
E = 160000
C = 128
K5 = 125

NC = 2
NS = 16
NW = NC * NS
NP = 10240
NPT = NP // NS
ZB = 16

CBL = 16
EPT = E // NW
NCHT = 316
EPTP = NCHT * CBL
NQUAD = NCHT // 4

PIPELINED = True

CBS = 64
NCH_S = E // CBS


def _basis_body(ei_ref, at_ref, gidx_ref, bw_ref):
    src = ei_ref[0, :]
    p0 = at_ref[0, :] * 4.0
    p1 = at_ref[1, :] * 4.0
    p2 = at_ref[2, :] * 4.0
    fl0 = jnp.floor(p0)
    fl1 = jnp.floor(p1)
    fl2 = jnp.floor(p2)
    fr = (p0 - fl0, p1 - fl1, p2 - fl2)
    fli = (fl0.astype(jnp.int32), fl1.astype(jnp.int32), fl2.astype(jnp.int32))
    off = (1, 5, 25)
    for s in range(8):
        b = None
        wi = None
        for d in range(3):
            bit = (s >> d) & 1
            v = fr[d] if bit == 1 else (1.0 - fr[d])
            b = v if b is None else b * v
            idx = (fli[d] + bit) % 5
            t = idx * off[d]
            wi = t if wi is None else wi + t
        gidx_ref[s, :] = wi * N + src
        bw_ref[s, :] = b


def _basis(edge_index, attr_t):
    return pl.pallas_call(
        _basis_body,
        out_shape=(
            jax.ShapeDtypeStruct((8, E), jnp.int32),
            jax.ShapeDtypeStruct((8, E), jnp.float32),
        ),
    )(edge_index, attr_t)


BN_N = 2000


def _ymm_body(x_ref, w_ref, y_ref):
    y_ref[0] = jnp.dot(x_ref[...], w_ref[0],
                       preferred_element_type=jnp.float32)


def _ymm(x, w):
    nt = N // BN_N
    y = pl.pallas_call(
        _ymm_body,
        grid=(nt, K5),
        in_specs=[
            pl.BlockSpec((BN_N, C), lambda n, k: (n, 0)),
            pl.BlockSpec((1, C, C), lambda n, k: (k, 0, 0)),
        ],
        out_specs=pl.BlockSpec((1, BN_N, C), lambda n, k: (k, n, 0)),
        out_shape=jax.ShapeDtypeStruct((K5, N, C), jnp.float32),
    )(x.astype(jnp.bfloat16), w.astype(jnp.bfloat16))
    return y.reshape(K5 * N, C)


def _zero_fill(zb_v, width):
    def _zrow(r, _):
        for cc in range(width // 16):
            zb_v[r, pl.ds(cc * 16, 16)] = jnp.zeros((16,), jnp.float32)
        return 0

    lax.fori_loop(0, ZB, _zrow, 0)


def _zero_spmem(zb_v, sh, s, width):
    for r in range(NPT // ZB):
        pltpu.sync_copy(zb_v.at[0:ZB, 0:width],
                        sh.at[pl.ds(s * NPT + r * ZB, ZB)])


HP = NP // NC
HPT = HP // NS


def _sc_short_body(x_hbm, src_hbm, dst_hbm, aggx_hbm, cnt_hbm,
                   src_v, dst_v, ldst_v, rows_v, ones_v, zb_v,
                   aggx_sh, cnt_sh, sem, semc):
    c = lax.axis_index("c")
    s = lax.axis_index("s")

    _zero_fill(zb_v, C)

    def _orow(r, _):
        for cc in range(8):
            ones_v[r, pl.ds(cc * 16, 16)] = jnp.ones((16,), jnp.float32)
        return 0

    lax.fori_loop(0, CBS, _orow, 0)

    for r in range(HPT // ZB):
        pltpu.sync_copy(zb_v.at[0:ZB, :],
                        aggx_sh.at[pl.ds(s * HPT + r * ZB, ZB)])
        pltpu.sync_copy(zb_v.at[0:ZB, :],
                        cnt_sh.at[pl.ds(s * HPT + r * ZB, ZB)])

    @pl.when(s == 0)
    def _():
        pltpu.sync_copy(zb_v.at[0:16, :], aggx_sh.at[pl.ds(HP, 16)])
        pltpu.sync_copy(zb_v.at[0:16, :], cnt_sh.at[pl.ds(HP, 16)])

    plsc.subcore_barrier()
    lo = c * HP

    def _zrows(r, _):
        for cc in range(8):
            rows_v[r, pl.ds(cc * 16, 16)] = jnp.zeros((16,), jnp.float32)
        return 0

    lax.fori_loop(0, CBS, _zrows, 0)
    for r in range(CBS // 16):
        ldst_v[pl.ds(r * 16, 16)] = jnp.zeros((16,), jnp.int32)
    pltpu.async_copy(rows_v, aggx_sh.at[ldst_v], semc, add=True)
    pltpu.async_copy(rows_v, cnt_sh.at[ldst_v], semc, add=True)

    def _chunk(j, _):
        cid = s + j * NS

        @pl.when(cid < NCH_S)
        def _():
            base = cid * CBS
            pltpu.sync_copy(src_hbm.at[pl.ds(base, CBS)], src_v)
            pltpu.sync_copy(dst_hbm.at[pl.ds(base, CBS)], dst_v)
            for r in range(CBS // 16):
                sl = pl.ds(r * 16, 16)
                d = dst_v[sl] - lo
                ok = (d >= 0) & (d < HP)
                ldst_v[sl] = jnp.where(ok, d, HP)
            pltpu.async_copy(x_hbm.at[src_v], rows_v, sem).wait()
            pltpu.make_async_copy(rows_v, aggx_sh.at[ldst_v], semc).wait()
            pltpu.make_async_copy(ones_v, cnt_sh.at[ldst_v], semc).wait()
            pltpu.async_copy(rows_v, aggx_sh.at[ldst_v], semc, add=True)
            pltpu.async_copy(ones_v, cnt_sh.at[ldst_v], semc, add=True)

        return 0

    lax.fori_loop(0, NIT_S2, _chunk, 0)
    pltpu.make_async_copy(rows_v, aggx_sh.at[ldst_v], semc).wait()
    pltpu.make_async_copy(ones_v, cnt_sh.at[ldst_v], semc).wait()
    plsc.subcore_barrier()

    pltpu.sync_copy(aggx_sh.at[pl.ds(s * HPT, HPT)],
                    aggx_hbm.at[c, pl.ds(s * HPT, HPT)])
    pltpu.sync_copy(cnt_sh.at[pl.ds(s * HPT, HPT)],
                    cnt_hbm.at[c, pl.ds(s * HPT, HPT)])


NIT_S2 = (NCH_S + NS - 1) // NS


def _sc_short(x, src, dst):
    mesh = plsc.VectorSubcoreMesh(core_axis_name="c", subcore_axis_name="s")
    f = functools.partial(
        pl.kernel,
        mesh=mesh,
        out_type=(
            jax.ShapeDtypeStruct((NC, HP, C), jnp.float32),
            jax.ShapeDtypeStruct((NC, HP, C), jnp.float32),
        ),
        scratch_types=[
            pltpu.VMEM((CBS,), jnp.int32),
            pltpu.VMEM((CBS,), jnp.int32),
            pltpu.VMEM((CBS,), jnp.int32),
            pltpu.VMEM((CBS, C), jnp.float32),
            pltpu.VMEM((CBS, C), jnp.float32),
            pltpu.VMEM((ZB, C), jnp.float32),
            pltpu.VMEM_SHARED((HP + 16, C), jnp.float32),
            pltpu.VMEM_SHARED((HP + 16, C), jnp.float32),
            pltpu.SemaphoreType.DMA,
            pltpu.SemaphoreType.DMA,
        ],
    )(_sc_short_body)
    return f(x, src, dst)


def _sc_layer_body(y_hbm, gidx_hbm, bw_hbm, dst_hbm, agg_hbm,
                   gidx_v, bw_v, dstm_v, rows_v, msg_v, zb_v, agg_sh,
                   semg0, semg1, semm0, semm1, sems0, sems1):
    c = lax.axis_index("c")
    s = lax.axis_index("s")
    wid = s * NC + c

    _zero_fill(zb_v, C)
    _zero_spmem(zb_v, agg_sh, s, C)
    plsc.subcore_barrier()

    gb = wid * (NCHT * 128)
    db = wid * (NCHT * 16)
    semg = (semg0, semg1)
    semm = (semm0, semm1)
    sems_s = (sems0, sems1)

    def _meta_copies(m, j0):
        return [
            pltpu.make_async_copy(gidx_hbm.at[pl.ds(gb + j0 * 128, 256)],
                                  gidx_v.at[m], semm[m]),
            pltpu.make_async_copy(bw_hbm.at[pl.ds(gb + j0 * 128, 128)],
                                  bw_v.at[m, 0], semm[m]),
            pltpu.make_async_copy(bw_hbm.at[pl.ds(gb + j0 * 128 + 128, 128)],
                                  bw_v.at[m, 1], semm[m]),
            pltpu.make_async_copy(dst_hbm.at[pl.ds(db + j0 * 16, 16)],
                                  dstm_v.at[m, 0], semm[m]),
            pltpu.make_async_copy(dst_hbm.at[pl.ds(db + j0 * 16 + 16, 16)],
                                  dstm_v.at[m, 1], semm[m]),
        ]

    def _meta_start(m, j0):
        for cp in _meta_copies(m, j0):
            cp.start()

    def _meta_wait(m):
        for cp in _meta_copies(m, 0):
            cp.wait()

    NSPL = 4

    def _gather_copies(m, jj, p):
        w = 128 // NSPL
        return [
            pltpu.make_async_copy(
                y_hbm.at[gidx_v.at[m, pl.ds(jj * 128 + i * w, w)]],
                rows_v.at[p, pl.ds(i * w, w)], semg[p])
            for i in range(NSPL)
        ]

    def _gather_start(m, jj, p):
        for cp in _gather_copies(m, jj, p):
            cp.start()

    def _gather_wait(m, jj, p):
        for cp in _gather_copies(m, jj, p):
            cp.wait()

    def _scatter_wait(p):
        pltpu.make_async_copy(msg_v.at[p], agg_sh.at[dstm_v.at[0, 0]],
                              sems_s[p]).wait()

    def _combine(m, jj, p):
        _scatter_wait(p)

        def _epair(e2, _):
            v = bw_v[m, jj, pl.ds(e2 * 16, 16)]
            for half in range(2):
                eb = e2 * 16 + half * 8
                b = [v[half * 8 + t] for t in range(8)]
                for cc in range(8):
                    sl = pl.ds(cc * 16, 16)
                    pr = [b[t] * rows_v[p, eb + t, sl] for t in range(8)]
                    acc = (((pr[0] + pr[1]) + (pr[2] + pr[3]))
                           + ((pr[4] + pr[5]) + (pr[6] + pr[7])))
                    msg_v[p, e2 * 2 + half, sl] = acc
            return 0

        lax.fori_loop(0, CBL // 2, _epair, 0)
        pltpu.async_copy(msg_v.at[p], agg_sh.at[dstm_v.at[m, jj]],
                         sems_s[p], add=True)

    def _quad(q, _):
        nx0 = lax.rem(4 * q + 4, NCHT)
        nx1 = lax.rem(4 * q + 6, NCHT)
        _gather_wait(0, 0, 0)
        _combine(0, 0, 0)
        _gather_start(1, 0, 0)
        _gather_wait(0, 1, 1)
        _combine(0, 1, 1)
        _meta_start(0, nx0)
        _gather_start(1, 1, 1)
        _gather_wait(1, 0, 0)
        _combine(1, 0, 0)
        _meta_wait(0)
        _gather_start(0, 0, 0)
        _gather_wait(1, 1, 1)
        _combine(1, 1, 1)
        _meta_start(1, nx1)
        _meta_wait(1)
        _gather_start(0, 1, 1)
        return 0

    def _quad_serial(q, _):
        nx0 = lax.rem(4 * q + 4, NCHT)
        nx1 = lax.rem(4 * q + 6, NCHT)
        for m, jj in ((0, 0), (0, 1), (1, 0), (1, 1)):
            _gather_start(m, jj, 0)
            _gather_wait(m, jj, 0)
            _combine(m, jj, 0)
        _meta_start(0, nx0)
        _meta_wait(0)
        _meta_start(1, nx1)
        _meta_wait(1)
        return 0

    _meta_start(0, 0)
    _meta_start(1, 2)
    _meta_wait(0)
    _meta_wait(1)
    pltpu.async_copy(zb_v, agg_sh.at[dstm_v.at[0, 0]], sems_s[0], add=True)
    pltpu.async_copy(zb_v, agg_sh.at[dstm_v.at[0, 0]], sems_s[1], add=True)
    if PIPELINED:
        _gather_start(0, 0, 0)
        _gather_start(0, 1, 1)
        lax.fori_loop(0, NQUAD, _quad, 0)
        _gather_wait(0, 0, 0)
        _gather_wait(0, 1, 1)
    else:
        lax.fori_loop(0, NQUAD, _quad_serial, 0)
    _scatter_wait(0)
    _scatter_wait(1)

    plsc.subcore_barrier()
    pltpu.sync_copy(agg_sh.at[pl.ds(s * NPT, NPT)],
                    agg_hbm.at[c, pl.ds(s * NPT, NPT)])


def _sc_layer(y, gidx_p, bw_p, dst_p):
    mesh = plsc.VectorSubcoreMesh(core_axis_name="c", subcore_axis_name="s")
    f = functools.partial(
        pl.kernel,
        mesh=mesh,
        out_type=jax.ShapeDtypeStruct((NC, NP, C), jnp.float32),
        scratch_types=[
            pltpu.VMEM((2, 256), jnp.int32),
            pltpu.VMEM((2, 2, 128), jnp.float32),
            pltpu.VMEM((2, 2, 16), jnp.int32),
            pltpu.VMEM((2, 128, C), jnp.float32),
            pltpu.VMEM((2, CBL, C), jnp.float32),
            pltpu.VMEM((ZB, C), jnp.float32),
            pltpu.VMEM_SHARED((NP, C), jnp.float32),
            pltpu.SemaphoreType.DMA,
            pltpu.SemaphoreType.DMA,
            pltpu.SemaphoreType.DMA,
            pltpu.SemaphoreType.DMA,
            pltpu.SemaphoreType.DMA,
            pltpu.SemaphoreType.DMA,
        ],
    )(_sc_layer_body)
    return f(y, gidx_p, bw_p, dst_p)


def _bn(pre, g, be):
    mu = jnp.mean(pre, axis=0)
    d = pre - mu
    var = jnp.mean(d * d, axis=0)
    return g * d / jnp.sqrt(var + 1e-5) + be


def _elu(v):
    return jnp.where(v > 0.0, v, jnp.exp(jnp.minimum(v, 0.0)) - 1.0)


def _cnt_full(cnt_ref):
    return jnp.concatenate([cnt_ref[0], cnt_ref[1]], axis=0)[0:N, :]


def _post1_body(agg_ref, cnt_ref, x_ref, root_ref, b_ref, g_ref, be_ref,
                h_ref):
    cnt = _cnt_full(cnt_ref)
    agg = (agg_ref[0][0:N, :] + agg_ref[1][0:N, :]) / jnp.maximum(cnt, 1.0)
    pre = agg + jnp.dot(x_ref[...], root_ref[...],
                        preferred_element_type=jnp.float32) + b_ref[...]
    h_ref[...] = _elu(_bn(pre, g_ref[...], be_ref[...]))


def _post1(agg, cnt, x, root, b, g, be):
    return pl.pallas_call(
        _post1_body,
        out_shape=jax.ShapeDtypeStruct((N, C), jnp.float32),
    )(agg, cnt, x, root, b, g, be)


def _post2_body(agg_ref, aggx_ref, cnt_ref, h_ref, x_ref,
                root2_ref, b2_ref, g2_ref, be2_ref,
                ws_ref, roots_ref, bs_ref, gs_ref, bes_ref, out_ref):
    cnt = jnp.maximum(_cnt_full(cnt_ref), 1.0)
    agg = (agg_ref[0][0:N, :] + agg_ref[1][0:N, :]) / cnt
    left_pre = agg + jnp.dot(h_ref[...], root2_ref[...],
                             preferred_element_type=jnp.float32) + b2_ref[...]
    left = _bn(left_pre, g2_ref[...], be2_ref[...])
    aggx = jnp.concatenate([aggx_ref[0], aggx_ref[1]], axis=0)[0:N, :] / cnt
    short_pre = (jnp.dot(aggx, ws_ref[...], preferred_element_type=jnp.float32)
                 + jnp.dot(x_ref[...], roots_ref[...],
                           preferred_element_type=jnp.float32) + bs_ref[...])
    short = _bn(short_pre, gs_ref[...], bes_ref[...])
    out_ref[...] = _elu(left + short)


def _post2(agg, aggx, cnt, h, x, root2, b2, g2, be2, ws, roots, bs, gs, bes):
    return pl.pallas_call(
        _post2_body,
        out_shape=jax.ShapeDtypeStruct((N, C), jnp.float32),
    )(agg, aggx, cnt, h, x, root2, b2, g2, be2, ws, roots, bs, gs, bes)


def kernel(x, edge_index, edge_attr, W1, root1, b1, g1, be1,
           W2, root2, b2, g2, be2, Ws, roots, bs, gs, bes):
    edge_index = edge_index.astype(jnp.int32)
    src = edge_index[0]
    dst = edge_index[1]
    attr_t = edge_attr.T

    gidx, bw = _basis(edge_index, attr_t)
    pad = ((0, 0), (0, EPTP - EPT), (0, 0))
    gidx_p = jnp.pad(gidx.T.reshape(NW, EPT, 8), pad).reshape(NW * NCHT * 128)
    bw_p = jnp.pad(bw.T.reshape(NW, EPT, 8), pad).reshape(NW * NCHT * 128)
    dst_p = jnp.pad(dst.reshape(NW, EPT), ((0, 0), (0, EPTP - EPT))
                    ).reshape(NW * NCHT * 16)

    y1 = _ymm(x, W1)
    aggx, cnt = _sc_short(x, src, dst)
    agg1 = _sc_layer(y1, gidx_p, bw_p, dst_p)
    h = _post1(agg1, cnt, x, root1, b1, g1, be1)
    y2 = _ymm(h, W2)
    agg2 = _sc_layer(y2, gidx_p, bw_p, dst_p)
    return _post2(agg2, aggx, cnt, h, x, root2, b2, g2, be2,
                  Ws[0], roots, bs, gs, bes)
